# baseline (speedup 1.0000x reference)
import functools

import jax
import jax.numpy as jnp
from jax import lax
from jax.experimental import pallas as pl
from jax.experimental.pallas import tpu as pltpu
from jax.experimental.pallas import tpu_sc as plsc

BATCH = 16384
FACTORS = 64
ROWS = 1000000
NUM_WORKERS = 32
ROWS_PER_W = BATCH // NUM_WORKERS
REGION = 31232
NCHUNK = REGION // 512
CW = 512
PITCH = CW + 1
HCAP = 1536
CCAP = 96
KB = FACTORS // 16
PAD_SLOT = BATCH

_mesh = plsc.VectorSubcoreMesh(core_axis_name="c", subcore_axis_name="s")


@functools.partial(
    pl.kernel,
    mesh=_mesh,
    out_type=jax.ShapeDtypeStruct((2, BATCH + 16, 2 * FACTORS), jnp.float32),
    scratch_types=[
        pltpu.VMEM((BATCH,), jnp.int32),
        pltpu.VMEM((FACTORS, PITCH), jnp.float32),
        pltpu.VMEM((HCAP,), jnp.int32),
        pltpu.VMEM((HCAP,), jnp.int32),
        pltpu.VMEM((CCAP,), jnp.int32),
        pltpu.VMEM((CCAP,), jnp.int32),
        pltpu.VMEM((16, 2 * FACTORS), jnp.float32),
        pltpu.VMEM((FACTORS, 64), jnp.float32),
        pltpu.SemaphoreType.DMA,
    ],
    compiler_params=pltpu.CompilerParams(needs_layout_passes=False),
)
def _sweep_sc(uidx_hbm, iidx_hbm, ut_hbm, it_hbm, utail_hbm, itail_hbm,
              out_hbm, ix_v, buf_v, hr_v, hs_v, cr_v, cs_v, stage_v,
              edge_v, sem):
    wid = lax.axis_index("s") * 2 + lax.axis_index("c")
    lo = wid * REGION
    is_last = wid == NUM_WORKERS - 1
    hi = jnp.where(is_last, ROWS, lo + REGION)
    iota16 = lax.iota(jnp.int32, 16)

    def one_table(t, tab_hbm, tail_hbm, idx_hbm):
        pltpu.sync_copy(idx_hbm, ix_v)

        def bin_body(i, pos):
            v = ix_v[pl.ds(i * 16, 16)]
            m = (v >= lo) & (v < hi)
            plsc.store_compressed(hr_v.at[pl.ds(pos, 16)], v, mask=m)
            plsc.store_compressed(hs_v.at[pl.ds(pos, 16)],
                                  i * 16 + iota16, mask=m)
            return pos + plsc.all_reduce_population_count(m)[0]

        nhits = lax.fori_loop(0, BATCH // 16, bin_body, 0)
        pad_r = jnp.full((16,), lo, jnp.int32)
        pad_s = jnp.full((16,), PAD_SLOT, jnp.int32)
        for j in range(2):
            hr_v[pl.ds(nhits + j * 16, 16)] = pad_r
            hs_v[pl.ds(nhits + j * 16, 16)] = pad_s

        def window(start, width, buf):
            if width == CW:
                pltpu.async_copy(tab_hbm.at[:, pl.ds(start, width)],
                                 buf.at[:, pl.ds(0, width)], sem).wait()
            else:
                pltpu.sync_copy(tail_hbm, buf)

            def scan_body(h, cpos):
                rv = hr_v[pl.ds(h * 16, 16)]
                sv = hs_v[pl.ds(h * 16, 16)]
                m = (rv >= start) & (rv < start + width)
                plsc.store_compressed(cr_v.at[pl.ds(cpos, 16)], rv - start,
                                      mask=m)
                plsc.store_compressed(cs_v.at[pl.ds(cpos, 16)], sv, mask=m)
                return cpos + plsc.all_reduce_population_count(m)[0]

            nscan = (nhits + 31) // 16
            cnt = lax.fori_loop(0, nscan, scan_body, 0)
            zero16 = jnp.zeros((16,), jnp.int32)
            for j in range(2):
                cr_v[pl.ds(cnt + j * 16, 16)] = zero16
                cs_v[pl.ds(cnt + j * 16, 16)] = pad_s

            def group_body(g, carry):
                rl = cr_v[pl.ds(g * 16, 16)]
                sl16 = cs_v[pl.ds(g * 16, 16)]
                for j in range(16):
                    rsp = rl.at[jnp.full((16,), j, jnp.int32)].get(
                        mode="promise_in_bounds")
                    for k in range(KB):
                        vals = plsc.load_gather(
                            buf, [k * 16 + iota16, rsp])
                        stage_v[j, pl.ds(k * 16, 16)] = vals
                pltpu.async_copy(stage_v, out_hbm.at[t].at[sl16],
                                 sem).wait()
                return carry

            lax.fori_loop(0, (cnt + 15) // 16, group_body, 0)

        def win_body(c, carry):
            window(lo + c * CW, CW, buf_v)
            return carry

        nwin = jnp.where(is_last, NCHUNK + 1, NCHUNK)
        lax.fori_loop(0, nwin, win_body, 0)

        @pl.when(is_last)
        def _edge():
            window(lo + (NCHUNK + 1) * CW, ROWS - (NUM_WORKERS - 1) * REGION
                   - (NCHUNK + 1) * CW, edge_v)

    one_table(0, ut_hbm, utail_hbm, uidx_hbm)
    one_table(1, it_hbm, itail_hbm, iidx_hbm)


@functools.partial(
    pl.kernel,
    mesh=_mesh,
    out_type=jax.ShapeDtypeStruct((BATCH,), jnp.float32),
    scratch_types=[
        pltpu.VMEM((2, 128, 2 * FACTORS), jnp.float32),
        pltpu.VMEM((2, 128, 2 * FACTORS), jnp.float32),
        pltpu.VMEM((FACTORS,), jnp.float32),
        pltpu.VMEM((16,), jnp.float32),
        pltpu.VMEM((ROWS_PER_W,), jnp.float32),
        pltpu.SemaphoreType.DMA,
        pltpu.SemaphoreType.DMA,
    ],
    compiler_params=pltpu.CompilerParams(needs_layout_passes=False),
)
def _dot_sc(rows_hbm, w_hbm, b_hbm, out_hbm,
            ur_v, ir_v, w_v, b_v, out_v, sem0, sem1):
    sems = (sem0, sem1)
    wid = lax.axis_index("s") * 2 + lax.axis_index("c")
    base = wid * ROWS_PER_W

    pltpu.sync_copy(w_hbm, w_v)
    pltpu.sync_copy(b_hbm, b_v)

    def start_chunk(c):
        buf = c % 2
        return (
            pltpu.async_copy(rows_hbm.at[0].at[pl.ds(base + c * 128, 128)],
                             ur_v.at[buf], sems[buf]),
            pltpu.async_copy(rows_hbm.at[1].at[pl.ds(base + c * 128, 128)],
                             ir_v.at[buf], sems[buf]),
        )

    iota16 = lax.iota(jnp.int32, 16)
    last_lane = iota16 == 15
    b_onehot = jnp.where(iota16 == 0, b_v[...], 0.0)
    wv = [w_v[pl.ds(k * 16, 16)] for k in range(KB)]

    inflight = {0: start_chunk(0)}
    for c in range(4):
        if c + 1 < 4:
            inflight[c + 1] = start_chunk(c + 1)
        for cp in inflight.pop(c):
            cp.wait()
        buf = c % 2
        ur_c = ur_v.at[buf]
        ir_c = ir_v.at[buf]

        def group_body(g, carry, c=c, ur_c=ur_c, ir_c=ir_c):
            for s in range(16):
                r = g * 16 + s
                acc = b_onehot
                for k in range(KB):
                    sl = pl.ds(k * 16, 16)
                    acc = acc + ur_c[r, sl] * ir_c[r, sl] * wv[k]
                tot = plsc.cumsum(acc)
                plsc.store_scatter(
                    out_v, [jnp.full((16,), c * 128, jnp.int32) + r],
                    tot, mask=last_lane)
            return carry

        lax.fori_loop(0, 8, group_body, 0)

    pltpu.sync_copy(out_v, out_hbm.at[pl.ds(base, ROWS_PER_W)])


def kernel(user_idx, item_idx, user_table, item_table, W, b):
    ut_t = user_table.T
    it_t = item_table.T
    utail = ut_t[:, (ROWS // CW) * CW:]
    itail = it_t[:, (ROWS // CW) * CW:]
    w = W.reshape(FACTORS)
    bvec = jnp.broadcast_to(b, (16,)).astype(jnp.float32)
    rows = _sweep_sc(user_idx, item_idx, ut_t, it_t, utail, itail)
    return _dot_sc(rows, w, bvec)


# speedup vs baseline: 1.5919x; 1.5919x over previous
import functools

import jax
import jax.numpy as jnp
from jax import lax
from jax.experimental import pallas as pl
from jax.experimental.pallas import tpu as pltpu
from jax.experimental.pallas import tpu_sc as plsc

BATCH = 16384
FACTORS = 64
ROWS = 1000000
NUM_WORKERS = 32
ROWS_PER_W = BATCH // NUM_WORKERS
REGION = 31744
CW = 1024
NCHUNK = REGION // CW
LAST_FULL = (1000000 - 31 * REGION) // CW
PITCH = CW + 1
HCAP = 1536
CCAP = 96
KB = FACTORS // 16
PAD_SLOT = BATCH

_mesh = plsc.VectorSubcoreMesh(core_axis_name="c", subcore_axis_name="s")


@functools.partial(
    pl.kernel,
    mesh=_mesh,
    out_type=jax.ShapeDtypeStruct((2, BATCH + 16, 2 * FACTORS), jnp.float32),
    scratch_types=[
        pltpu.VMEM((BATCH,), jnp.int32),
        pltpu.VMEM((FACTORS, PITCH), jnp.float32),
        pltpu.VMEM((HCAP,), jnp.int32),
        pltpu.VMEM((HCAP,), jnp.int32),
        pltpu.VMEM((CCAP,), jnp.int32),
        pltpu.VMEM((CCAP,), jnp.int32),
        pltpu.VMEM((16, 2 * FACTORS), jnp.float32),
        pltpu.VMEM((FACTORS, 64), jnp.float32),
        pltpu.SemaphoreType.DMA,
    ],
    compiler_params=pltpu.CompilerParams(needs_layout_passes=False),
)
def _sweep_sc(uidx_hbm, iidx_hbm, ut_hbm, it_hbm, utail_hbm, itail_hbm,
              out_hbm, ix_v, buf_v, hr_v, hs_v, cr_v, cs_v, stage_v,
              edge_v, sem):
    wid = lax.axis_index("s") * 2 + lax.axis_index("c")
    lo = wid * REGION
    is_last = wid == NUM_WORKERS - 1
    hi = jnp.where(is_last, ROWS, lo + REGION)
    iota16 = lax.iota(jnp.int32, 16)

    def one_table(t, tab_hbm, tail_hbm, idx_hbm):
        pltpu.sync_copy(idx_hbm, ix_v)

        def bin_body(i, pos):
            v = ix_v[pl.ds(i * 16, 16)]
            m = (v >= lo) & (v < hi)
            plsc.store_compressed(hr_v.at[pl.ds(pos, 16)], v, mask=m)
            plsc.store_compressed(hs_v.at[pl.ds(pos, 16)],
                                  i * 16 + iota16, mask=m)
            return pos + plsc.all_reduce_population_count(m)[0]

        nhits = lax.fori_loop(0, BATCH // 16, bin_body, 0)
        pad_r = jnp.full((16,), lo, jnp.int32)
        pad_s = jnp.full((16,), PAD_SLOT, jnp.int32)
        for j in range(2):
            hr_v[pl.ds(nhits + j * 16, 16)] = pad_r
            hs_v[pl.ds(nhits + j * 16, 16)] = pad_s

        def window(start, width, buf):
            if width == 64:
                pltpu.sync_copy(tail_hbm, buf)
            else:
                pltpu.async_copy(tab_hbm.at[:, pl.ds(start, width)],
                                 buf.at[:, pl.ds(0, width)], sem).wait()

            def scan_body(h, cpos):
                rv = hr_v[pl.ds(h * 16, 16)]
                sv = hs_v[pl.ds(h * 16, 16)]
                m = (rv >= start) & (rv < start + width)
                plsc.store_compressed(cr_v.at[pl.ds(cpos, 16)], rv - start,
                                      mask=m)
                plsc.store_compressed(cs_v.at[pl.ds(cpos, 16)], sv, mask=m)
                return cpos + plsc.all_reduce_population_count(m)[0]

            nscan = (nhits + 31) // 16
            cnt = lax.fori_loop(0, nscan, scan_body, 0)
            zero16 = jnp.zeros((16,), jnp.int32)
            for j in range(2):
                cr_v[pl.ds(cnt + j * 16, 16)] = zero16
                cs_v[pl.ds(cnt + j * 16, 16)] = pad_s

            def group_body(g, carry):
                rl = cr_v[pl.ds(g * 16, 16)]
                sl16 = cs_v[pl.ds(g * 16, 16)]
                for j in range(16):
                    rsp = rl.at[jnp.full((16,), j, jnp.int32)].get(
                        mode="promise_in_bounds")
                    for k in range(KB):
                        vals = plsc.load_gather(
                            buf, [k * 16 + iota16, rsp])
                        stage_v[j, pl.ds(k * 16, 16)] = vals
                pltpu.async_copy(stage_v, out_hbm.at[t].at[sl16],
                                 sem).wait()
                return carry

            lax.fori_loop(0, (cnt + 15) // 16, group_body, 0)

        def win_body(c, carry):
            window(lo + c * CW, CW, buf_v)
            return carry

        nwin = jnp.where(is_last, LAST_FULL, NCHUNK)
        lax.fori_loop(0, nwin, win_body, 0)

        @pl.when(is_last)
        def _edge():
            window(lo + LAST_FULL * CW, 512, buf_v)
            window(lo + LAST_FULL * CW + 512, 64, edge_v)

    one_table(0, ut_hbm, utail_hbm, uidx_hbm)
    one_table(1, it_hbm, itail_hbm, iidx_hbm)


@functools.partial(
    pl.kernel,
    mesh=_mesh,
    out_type=jax.ShapeDtypeStruct((BATCH,), jnp.float32),
    scratch_types=[
        pltpu.VMEM((2, 128, 2 * FACTORS), jnp.float32),
        pltpu.VMEM((2, 128, 2 * FACTORS), jnp.float32),
        pltpu.VMEM((FACTORS,), jnp.float32),
        pltpu.VMEM((16,), jnp.float32),
        pltpu.VMEM((ROWS_PER_W,), jnp.float32),
        pltpu.SemaphoreType.DMA,
        pltpu.SemaphoreType.DMA,
    ],
    compiler_params=pltpu.CompilerParams(needs_layout_passes=False),
)
def _dot_sc(rows_hbm, w_hbm, b_hbm, out_hbm,
            ur_v, ir_v, w_v, b_v, out_v, sem0, sem1):
    sems = (sem0, sem1)
    wid = lax.axis_index("s") * 2 + lax.axis_index("c")
    base = wid * ROWS_PER_W

    pltpu.sync_copy(w_hbm, w_v)
    pltpu.sync_copy(b_hbm, b_v)

    def start_chunk(c):
        buf = c % 2
        return (
            pltpu.async_copy(rows_hbm.at[0].at[pl.ds(base + c * 128, 128)],
                             ur_v.at[buf], sems[buf]),
            pltpu.async_copy(rows_hbm.at[1].at[pl.ds(base + c * 128, 128)],
                             ir_v.at[buf], sems[buf]),
        )

    iota16 = lax.iota(jnp.int32, 16)
    last_lane = iota16 == 15
    b_onehot = jnp.where(iota16 == 0, b_v[...], 0.0)
    wv = [w_v[pl.ds(k * 16, 16)] for k in range(KB)]

    inflight = {0: start_chunk(0)}
    for c in range(4):
        if c + 1 < 4:
            inflight[c + 1] = start_chunk(c + 1)
        for cp in inflight.pop(c):
            cp.wait()
        buf = c % 2
        ur_c = ur_v.at[buf]
        ir_c = ir_v.at[buf]

        def group_body(g, carry, c=c, ur_c=ur_c, ir_c=ir_c):
            for s in range(16):
                r = g * 16 + s
                acc = b_onehot
                for k in range(KB):
                    sl = pl.ds(k * 16, 16)
                    acc = acc + ur_c[r, sl] * ir_c[r, sl] * wv[k]
                tot = plsc.cumsum(acc)
                plsc.store_scatter(
                    out_v, [jnp.full((16,), c * 128, jnp.int32) + r],
                    tot, mask=last_lane)
            return carry

        lax.fori_loop(0, 8, group_body, 0)

    pltpu.sync_copy(out_v, out_hbm.at[pl.ds(base, ROWS_PER_W)])


def kernel(user_idx, item_idx, user_table, item_table, W, b):
    ut_t = user_table.T
    it_t = item_table.T
    utail = ut_t[:, (ROWS // 128) * 128:]
    itail = it_t[:, (ROWS // 128) * 128:]
    w = W.reshape(FACTORS)
    bvec = jnp.broadcast_to(b, (16,)).astype(jnp.float32)
    rows = _sweep_sc(user_idx, item_idx, ut_t, it_t, utail, itail)
    return _dot_sc(rows, w, bvec)


# speedup vs baseline: 1.6090x; 1.0108x over previous
import functools

import jax
import jax.numpy as jnp
from jax import lax
from jax.experimental import pallas as pl
from jax.experimental.pallas import tpu as pltpu
from jax.experimental.pallas import tpu_sc as plsc

BATCH = 16384
FACTORS = 64
ROWS = 1000000
NUM_WORKERS = 32
ROWS_PER_W = BATCH // NUM_WORKERS
REGION = 31488
CW = 768
NCHUNK = REGION // CW
LAST_FULL = (ROWS - 31 * REGION) // CW
HCAP = 1024
CCAP = 96
KB = FACTORS // 16
PAD_SLOT = BATCH

_mesh = plsc.VectorSubcoreMesh(core_axis_name="c", subcore_axis_name="s")


@functools.partial(
    pl.kernel,
    mesh=_mesh,
    out_type=jax.ShapeDtypeStruct((2, BATCH + 16, 2 * FACTORS), jnp.float32),
    scratch_types=[
        pltpu.VMEM((BATCH,), jnp.int32),
        pltpu.VMEM((2, FACTORS, CW), jnp.float32),
        pltpu.VMEM((HCAP,), jnp.int32),
        pltpu.VMEM((HCAP,), jnp.int32),
        pltpu.VMEM((CCAP,), jnp.int32),
        pltpu.VMEM((CCAP,), jnp.int32),
        pltpu.VMEM((16, 2 * FACTORS), jnp.float32),
        pltpu.VMEM((FACTORS, 64), jnp.float32),
        pltpu.SemaphoreType.DMA,
        pltpu.SemaphoreType.DMA,
    ],
    compiler_params=pltpu.CompilerParams(needs_layout_passes=False),
)
def _sweep_sc(uidx_hbm, iidx_hbm, ut_hbm, it_hbm, utail_hbm, itail_hbm,
              out_hbm, ix_v, buf_v, hr_v, hs_v, cr_v, cs_v, stage_v,
              edge_v, sem0, sem1):
    sems = (sem0, sem1)
    wid = lax.axis_index("s") * 2 + lax.axis_index("c")
    lo = wid * REGION
    is_last = wid == NUM_WORKERS - 1
    hi = jnp.where(is_last, ROWS, lo + REGION)
    iota16 = lax.iota(jnp.int32, 16)

    def one_table(t, tab_hbm, tail_hbm, idx_hbm):
        pltpu.sync_copy(idx_hbm, ix_v)

        def bin_body(i, pos):
            v = ix_v[pl.ds(i * 16, 16)]
            m = (v >= lo) & (v < hi)
            plsc.store_compressed(hr_v.at[pl.ds(pos, 16)], v, mask=m)
            plsc.store_compressed(hs_v.at[pl.ds(pos, 16)],
                                  i * 16 + iota16, mask=m)
            return pos + plsc.all_reduce_population_count(m)[0]

        nhits = lax.fori_loop(0, BATCH // 16, bin_body, 0)
        pad_r = jnp.full((16,), lo, jnp.int32)
        pad_s = jnp.full((16,), PAD_SLOT, jnp.int32)
        for j in range(2):
            hr_v[pl.ds(nhits + j * 16, 16)] = pad_r
            hs_v[pl.ds(nhits + j * 16, 16)] = pad_s
        nscan = (nhits + 31) // 16

        def process(start, width, buf):
            def scan_body(h, cpos):
                rv = hr_v[pl.ds(h * 16, 16)]
                sv = hs_v[pl.ds(h * 16, 16)]
                m = (rv >= start) & (rv < start + width)
                plsc.store_compressed(cr_v.at[pl.ds(cpos, 16)], rv - start,
                                      mask=m)
                plsc.store_compressed(cs_v.at[pl.ds(cpos, 16)], sv, mask=m)
                return cpos + plsc.all_reduce_population_count(m)[0]

            cnt = lax.fori_loop(0, nscan, scan_body, 0)
            zero16 = jnp.zeros((16,), jnp.int32)
            for j in range(2):
                cr_v[pl.ds(cnt + j * 16, 16)] = zero16
                cs_v[pl.ds(cnt + j * 16, 16)] = pad_s

            def group_body(g, carry):
                rl = cr_v[pl.ds(g * 16, 16)]
                sl16 = cs_v[pl.ds(g * 16, 16)]
                for j in range(16):
                    rsp = rl.at[jnp.full((16,), j, jnp.int32)].get(
                        mode="promise_in_bounds")
                    for k in range(KB):
                        vals = plsc.load_gather(
                            buf, [k * 16 + iota16, rsp])
                        stage_v[j, pl.ds(k * 16, 16)] = vals
                pltpu.async_copy(stage_v, out_hbm.at[t].at[sl16],
                                 sems[0]).wait()
                return carry

            lax.fori_loop(0, (cnt + 15) // 16, group_body, 0)

        def start_w(c, b):
            pltpu.async_copy(tab_hbm.at[:, pl.ds(lo + c * CW, CW)],
                             buf_v.at[b], sems[1])

        def wait_w(c, b):
            pltpu.make_async_copy(tab_hbm.at[:, pl.ds(lo + c * CW, CW)],
                                  buf_v.at[b], sems[1]).wait()

        nwin = jnp.where(is_last, LAST_FULL, NCHUNK)

        @pl.when(nwin > 0)
        def _prime():
            start_w(0, 0)

        def win_pair(i, carry):
            c0 = i * 2

            @pl.when(c0 + 1 < nwin)
            def _s1():
                start_w(c0 + 1, 1)

            wait_w(c0, 0)
            process(lo + c0 * CW, CW, buf_v.at[0])

            @pl.when(c0 + 2 < nwin)
            def _s2():
                start_w(c0 + 2, 0)

            @pl.when(c0 + 1 < nwin)
            def _p1():
                wait_w(c0 + 1, 1)
                process(lo + (c0 + 1) * CW, CW, buf_v.at[1])

            return carry

        lax.fori_loop(0, (nwin + 1) // 2, win_pair, 0)

        @pl.when(is_last)
        def _edge():
            pltpu.sync_copy(tail_hbm, edge_v)
            process(lo + LAST_FULL * CW, 64, edge_v)

    one_table(0, ut_hbm, utail_hbm, uidx_hbm)
    one_table(1, it_hbm, itail_hbm, iidx_hbm)


@functools.partial(
    pl.kernel,
    mesh=_mesh,
    out_type=jax.ShapeDtypeStruct((BATCH,), jnp.float32),
    scratch_types=[
        pltpu.VMEM((2, 128, 2 * FACTORS), jnp.float32),
        pltpu.VMEM((2, 128, 2 * FACTORS), jnp.float32),
        pltpu.VMEM((FACTORS,), jnp.float32),
        pltpu.VMEM((16,), jnp.float32),
        pltpu.VMEM((ROWS_PER_W,), jnp.float32),
        pltpu.SemaphoreType.DMA,
        pltpu.SemaphoreType.DMA,
    ],
    compiler_params=pltpu.CompilerParams(needs_layout_passes=False),
)
def _dot_sc(rows_hbm, w_hbm, b_hbm, out_hbm,
            ur_v, ir_v, w_v, b_v, out_v, sem0, sem1):
    sems = (sem0, sem1)
    wid = lax.axis_index("s") * 2 + lax.axis_index("c")
    base = wid * ROWS_PER_W

    pltpu.sync_copy(w_hbm, w_v)
    pltpu.sync_copy(b_hbm, b_v)

    def start_chunk(c):
        buf = c % 2
        return (
            pltpu.async_copy(rows_hbm.at[0].at[pl.ds(base + c * 128, 128)],
                             ur_v.at[buf], sems[buf]),
            pltpu.async_copy(rows_hbm.at[1].at[pl.ds(base + c * 128, 128)],
                             ir_v.at[buf], sems[buf]),
        )

    iota16 = lax.iota(jnp.int32, 16)
    last_lane = iota16 == 15
    b_onehot = jnp.where(iota16 == 0, b_v[...], 0.0)
    wv = [w_v[pl.ds(k * 16, 16)] for k in range(KB)]

    inflight = {0: start_chunk(0)}
    for c in range(4):
        if c + 1 < 4:
            inflight[c + 1] = start_chunk(c + 1)
        for cp in inflight.pop(c):
            cp.wait()
        buf = c % 2
        ur_c = ur_v.at[buf]
        ir_c = ir_v.at[buf]

        def group_body(g, carry, c=c, ur_c=ur_c, ir_c=ir_c):
            for s in range(16):
                r = g * 16 + s
                acc = b_onehot
                for k in range(KB):
                    sl = pl.ds(k * 16, 16)
                    acc = acc + ur_c[r, sl] * ir_c[r, sl] * wv[k]
                tot = plsc.cumsum(acc)
                plsc.store_scatter(
                    out_v, [jnp.full((16,), c * 128, jnp.int32) + r],
                    tot, mask=last_lane)
            return carry

        lax.fori_loop(0, 8, group_body, 0)

    pltpu.sync_copy(out_v, out_hbm.at[pl.ds(base, ROWS_PER_W)])


def kernel(user_idx, item_idx, user_table, item_table, W, b):
    ut_t = user_table.T
    it_t = item_table.T
    utail = ut_t[:, (ROWS // 128) * 128:]
    itail = it_t[:, (ROWS // 128) * 128:]
    w = W.reshape(FACTORS)
    bvec = jnp.broadcast_to(b, (16,)).astype(jnp.float32)
    rows = _sweep_sc(user_idx, item_idx, ut_t, it_t, utail, itail)
    return _dot_sc(rows, w, bvec)
